# initial kernel scaffold (unmeasured)
import jax
import jax.numpy as jnp
from jax import lax
from jax.experimental import pallas as pl
from jax.experimental.pallas import tpu as pltpu

N_DEV = 8


def kernel(x, w_mat):
    m_per, k = x.shape
    _, n = w_mat.shape
    n_per = n // N_DEV
    m_tot = m_per * N_DEV

    def body(x_ref, w_ref, out_ref, send_buf, comm_buf, ax_send, ax_comm,
             send_sems, recv_sems, ax_send_sems, ax_recv_sems):
        my_i = lax.axis_index("i")

        barrier_sem = pltpu.get_barrier_semaphore()
        for off in range(1, N_DEV):
            pl.semaphore_signal(
                barrier_sem, inc=1,
                device_id=((my_i + off) % N_DEV,),
                device_id_type=pl.DeviceIdType.MESH,
            )
        pl.semaphore_wait(barrier_sem, N_DEV - 1)

        y = lax.dot_general(
            x_ref[...], w_ref[...],
            dimension_numbers=(((1,), (0,)), ((), ())),
            preferred_element_type=jnp.float32,
        )
        y = jnp.maximum(y, 0.0)
        local_amax = jnp.max(y)
        ax_send[...] = jnp.full((1, 128), local_amax, jnp.float32)

        for j in range(N_DEV):
            send_buf[j] = y[:, j * n_per:(j + 1) * n_per].astype(jnp.bfloat16)

        for j in range(N_DEV):
            @pl.when(j != my_i)
            def _(j=j):
                pltpu.make_async_remote_copy(
                    src_ref=send_buf.at[j],
                    dst_ref=comm_buf.at[my_i],
                    send_sem=send_sems.at[j],
                    recv_sem=recv_sems.at[my_i],
                    device_id=(j,),
                    device_id_type=pl.DeviceIdType.MESH,
                ).start()
                pltpu.make_async_remote_copy(
                    src_ref=ax_send.at[0],
                    dst_ref=ax_comm.at[my_i],
                    send_sem=ax_send_sems.at[j],
                    recv_sem=ax_recv_sems.at[my_i],
                    device_id=(j,),
                    device_id_type=pl.DeviceIdType.MESH,
                ).start()

        for j in range(N_DEV):
            @pl.when(j != my_i)
            def _(j=j):
                pltpu.make_async_remote_copy(
                    src_ref=send_buf.at[j], dst_ref=comm_buf.at[j],
                    send_sem=send_sems.at[j], recv_sem=recv_sems.at[j],
                    device_id=(j,), device_id_type=pl.DeviceIdType.MESH,
                ).wait_recv()
                pltpu.make_async_remote_copy(
                    src_ref=ax_send.at[0], dst_ref=ax_comm.at[j],
                    send_sem=ax_send_sems.at[j], recv_sem=ax_recv_sems.at[j],
                    device_id=(j,), device_id_type=pl.DeviceIdType.MESH,
                ).wait_recv()

        g_amax = local_amax
        for j in range(N_DEV):
            g_amax = jnp.maximum(
                g_amax, jnp.where(j == my_i, local_amax, ax_comm[j, 0]))

        inv_scale = 127.0 / g_amax
        scale = g_amax / 127.0

        def quant(c_bf16):
            cf = c_bf16.astype(jnp.float32)
            q = jnp.clip(jnp.round(cf * inv_scale), 0.0, 127.0)
            return q * scale

        for j in range(N_DEV):
            @pl.when(j == my_i)
            def _(j=j):
                out_ref[j * m_per:(j + 1) * m_per, :] = quant(send_buf[j])
            @pl.when(j != my_i)
            def _(j=j):
                out_ref[j * m_per:(j + 1) * m_per, :] = quant(comm_buf[j])

        for j in range(N_DEV):
            @pl.when(j != my_i)
            def _(j=j):
                pltpu.make_async_remote_copy(
                    src_ref=send_buf.at[j], dst_ref=comm_buf.at[j],
                    send_sem=send_sems.at[j], recv_sem=recv_sems.at[j],
                    device_id=(j,), device_id_type=pl.DeviceIdType.MESH,
                ).wait_send()
                pltpu.make_async_remote_copy(
                    src_ref=ax_send.at[0], dst_ref=ax_comm.at[j],
                    send_sem=ax_send_sems.at[j], recv_sem=ax_recv_sems.at[j],
                    device_id=(j,), device_id_type=pl.DeviceIdType.MESH,
                ).wait_send()

    return pl.pallas_call(
        body,
        out_shape=jax.ShapeDtypeStruct((m_tot, n_per), jnp.float32),
        in_specs=[
            pl.BlockSpec(memory_space=pltpu.VMEM),
            pl.BlockSpec(memory_space=pltpu.VMEM),
        ],
        out_specs=pl.BlockSpec(memory_space=pltpu.VMEM),
        scratch_shapes=[
            pltpu.VMEM((N_DEV, m_per, n_per), jnp.bfloat16),
            pltpu.VMEM((N_DEV, m_per, n_per), jnp.bfloat16),
            pltpu.VMEM((1, 128), jnp.float32),
            pltpu.VMEM((N_DEV, 128), jnp.float32),
            pltpu.SemaphoreType.DMA((N_DEV,)),
            pltpu.SemaphoreType.DMA((N_DEV,)),
            pltpu.SemaphoreType.DMA((N_DEV,)),
            pltpu.SemaphoreType.DMA((N_DEV,)),
        ],
        compiler_params=pltpu.CompilerParams(collective_id=0),
    )(x, w_mat)


# baseline (device time: 34970 ns/iter reference)
import jax
import jax.numpy as jnp
from jax import lax
from jax.experimental import pallas as pl
from jax.experimental.pallas import tpu as pltpu

N_DEV = 8


def kernel(x, w_mat):
    m_per, k = x.shape
    _, n = w_mat.shape
    n_per = n // N_DEV
    m_tot = m_per * N_DEV

    def body(x_ref, w_ref, out_ref, w_stage, send_buf, comm_buf, ax_send,
             ax_comm, w_sems, send_sems, recv_sems, ax_send_sems,
             ax_recv_sems):
        my_i = lax.axis_index("i")

        barrier_sem = pltpu.get_barrier_semaphore()
        for off in range(1, N_DEV):
            pl.semaphore_signal(
                barrier_sem, inc=1,
                device_id=((my_i + off) % N_DEV,),
                device_id_type=pl.DeviceIdType.MESH,
            )

        def chunk_of(t):
            return (my_i + 1 + t) % N_DEV

        def w_dma(t):
            c = chunk_of(t)
            return pltpu.make_async_copy(
                w_ref.at[:, pl.ds(c * n_per, n_per)],
                w_stage.at[t % 2],
                w_sems.at[t % 2],
            )

        w_dma(0).start()
        pl.semaphore_wait(barrier_sem, N_DEV - 1)

        local_amax = jnp.float32(0.0)
        for t in range(N_DEV):
            w_dma(t).wait()
            if t + 1 < N_DEV:
                w_dma(t + 1).start()
            y = lax.dot_general(
                x_ref[...], w_stage[t % 2],
                dimension_numbers=(((1,), (0,)), ((), ())),
                preferred_element_type=jnp.float32,
                precision=lax.Precision.DEFAULT,
            )
            y = jnp.maximum(y, 0.0)
            local_amax = jnp.maximum(local_amax, jnp.max(y))
            send_buf[t] = y.astype(jnp.bfloat16)
            if t < N_DEV - 1:
                pltpu.make_async_remote_copy(
                    src_ref=send_buf.at[t],
                    dst_ref=comm_buf.at[my_i],
                    send_sem=send_sems.at[t],
                    recv_sem=recv_sems.at[my_i],
                    device_id=(chunk_of(t),),
                    device_id_type=pl.DeviceIdType.MESH,
                ).start()

        ax_send[...] = jnp.full((1, 128), local_amax, jnp.float32)
        for t in range(N_DEV - 1):
            pltpu.make_async_remote_copy(
                src_ref=ax_send.at[0],
                dst_ref=ax_comm.at[my_i],
                send_sem=ax_send_sems.at[t],
                recv_sem=ax_recv_sems.at[my_i],
                device_id=(chunk_of(t),),
                device_id_type=pl.DeviceIdType.MESH,
            ).start()

        for s in range(N_DEV):
            @pl.when(s != my_i)
            def _(s=s):
                pltpu.make_async_remote_copy(
                    src_ref=send_buf.at[0], dst_ref=comm_buf.at[s],
                    send_sem=send_sems.at[s], recv_sem=recv_sems.at[s],
                    device_id=(s,), device_id_type=pl.DeviceIdType.MESH,
                ).wait_recv()
                pltpu.make_async_remote_copy(
                    src_ref=ax_send.at[0], dst_ref=ax_comm.at[s],
                    send_sem=ax_send_sems.at[s], recv_sem=ax_recv_sems.at[s],
                    device_id=(s,), device_id_type=pl.DeviceIdType.MESH,
                ).wait_recv()

        g_amax = local_amax
        for s in range(N_DEV):
            g_amax = jnp.maximum(
                g_amax, jnp.where(s == my_i, local_amax, ax_comm[s, 0]))

        inv_scale = 127.0 / g_amax
        scale = g_amax / 127.0

        def quant(c_bf16):
            cf = c_bf16.astype(jnp.float32)
            q = jnp.clip(jnp.round(cf * inv_scale), 0.0, 127.0)
            return q * scale

        out_ref[pl.ds(my_i * m_per, m_per), :] = quant(send_buf[N_DEV - 1])
        for s in range(N_DEV):
            @pl.when(s != my_i)
            def _(s=s):
                out_ref[s * m_per:(s + 1) * m_per, :] = quant(comm_buf[s])

        for t in range(N_DEV - 1):
            pltpu.make_async_remote_copy(
                src_ref=send_buf.at[t], dst_ref=comm_buf.at[0],
                send_sem=send_sems.at[t], recv_sem=recv_sems.at[t],
                device_id=(0,), device_id_type=pl.DeviceIdType.MESH,
            ).wait_send()
            pltpu.make_async_remote_copy(
                src_ref=ax_send.at[0], dst_ref=ax_comm.at[0],
                send_sem=ax_send_sems.at[t], recv_sem=ax_recv_sems.at[t],
                device_id=(0,), device_id_type=pl.DeviceIdType.MESH,
            ).wait_send()

    return pl.pallas_call(
        body,
        out_shape=jax.ShapeDtypeStruct((m_tot, n_per), jnp.float32),
        in_specs=[
            pl.BlockSpec(memory_space=pltpu.VMEM),
            pl.BlockSpec(memory_space=pl.ANY),
        ],
        out_specs=pl.BlockSpec(memory_space=pltpu.VMEM),
        scratch_shapes=[
            pltpu.VMEM((2, k, n_per), jnp.float32),
            pltpu.VMEM((N_DEV, m_per, n_per), jnp.bfloat16),
            pltpu.VMEM((N_DEV, m_per, n_per), jnp.bfloat16),
            pltpu.VMEM((1, 128), jnp.float32),
            pltpu.VMEM((N_DEV, 128), jnp.float32),
            pltpu.SemaphoreType.DMA((2,)),
            pltpu.SemaphoreType.DMA((N_DEV,)),
            pltpu.SemaphoreType.DMA((N_DEV,)),
            pltpu.SemaphoreType.DMA((N_DEV,)),
            pltpu.SemaphoreType.DMA((N_DEV,)),
        ],
        compiler_params=pltpu.CompilerParams(collective_id=0),
    )(x, w_mat)
